# baseline (speedup 1.0000x reference)
import jax
import jax.numpy as jnp
from jax.experimental import pallas as pl

_BB = 1024


def _bag_matmul_kernel(xa_ref, xb_ref, e_ref, o_ref):
    h = _BB // 2
    o_ref[:h, :] = jnp.dot(xa_ref[...], e_ref[...],
                           preferred_element_type=jnp.float32)
    o_ref[h:, :] = jnp.dot(xb_ref[...], e_ref[...],
                           preferred_element_type=jnp.float32)


def kernel(inputs, embeddings):
    B, V = inputs.shape
    _, D = embeddings.shape
    h = _BB // 2
    return pl.pallas_call(
        _bag_matmul_kernel,
        grid=(B // _BB,),
        in_specs=[
            pl.BlockSpec((h, V), lambda i: (2 * i, 0)),
            pl.BlockSpec((h, V), lambda i: (2 * i + 1, 0)),
            pl.BlockSpec((V, D), lambda i: (0, 0)),
        ],
        out_specs=pl.BlockSpec((_BB, D), lambda i: (i, 0)),
        out_shape=jax.ShapeDtypeStruct((B, D), jnp.float32),
    )(inputs, inputs, embeddings)


# speedup vs baseline: 1.0278x; 1.0278x over previous
import jax
import jax.numpy as jnp
from jax.experimental import pallas as pl

_BB = 512
_NS = 4


def _bag_matmul_kernel(*refs):
    xs = refs[:_NS]
    e_ref = refs[_NS]
    o_ref = refs[_NS + 1]
    h = _BB // _NS
    for k in range(_NS):
        o_ref[k * h:(k + 1) * h, :] = jnp.dot(
            xs[k][...], e_ref[...], preferred_element_type=jnp.float32)


def kernel(inputs, embeddings):
    B, V = inputs.shape
    _, D = embeddings.shape
    h = _BB // _NS
    def mk(k):
        return pl.BlockSpec((h, V), lambda i, k=k: (_NS * i + k, 0))
    return pl.pallas_call(
        _bag_matmul_kernel,
        grid=(B // _BB,),
        in_specs=[mk(k) for k in range(_NS)] + [
            pl.BlockSpec((V, D), lambda i: (0, 0)),
        ],
        out_specs=pl.BlockSpec((_BB, D), lambda i: (i, 0)),
        out_shape=jax.ShapeDtypeStruct((B, D), jnp.float32),
    )(*([inputs] * _NS), embeddings)
